# Initial kernel scaffold; baseline (speedup 1.0000x reference)
import jax
import jax.numpy as jnp
from jax.experimental import pallas as pl


def kernel(xyz1, xyz2, points1, points2, W1, b1, gamma1, beta1, W2, b2):
    raise NotImplementedError("write your pallas kernel here")


# speedup vs baseline: 23.6680x; 23.6680x over previous
import jax
import jax.numpy as jnp
from jax import lax
from jax.experimental import pallas as pl

_B, _N, _S, _D1, _D2 = 16, 4096, 1024, 256, 512
_CIN = _D1 + _D2
_M0, _M1 = 512, 512
_BLK = 512
_NB = _N // _BLK


def _stage1_body(x2t_ref, x1_ref, p2_ref, p1_ref, w1a_ref, w1b_ref, b1_ref,
                 h_ref, sum_ref, sq_ref):
    x2t = x2t_ref[0]
    x1b = x1_ref[0]
    d = jnp.dot(x2t, x1b, preferred_element_type=jnp.float32) * (-2.0)
    d = d + jnp.sum(x1b * x1b, axis=0, keepdims=True)
    d = d + jnp.sum(x2t * x2t, axis=1, keepdims=True)

    iota0 = lax.broadcasted_iota(jnp.int32, (_S, _BLK), 0)
    recs, sels = [], []
    rec_sum = jnp.zeros((1, _BLK), jnp.float32)
    for _ in range(3):
        mval = jnp.min(d, axis=0, keepdims=True)
        midx = jnp.min(jnp.where(d == mval, iota0, _S), axis=0,
                       keepdims=True)
        sel = iota0 == midx
        d = jnp.where(sel, jnp.float32(jnp.inf), d)
        r = 1.0 / (mval + 1e-8)
        recs.append(r)
        sels.append(sel)
        rec_sum = rec_sum + r
    wgt = jnp.where(sels[0], recs[0], 0.0)
    wgt = wgt + jnp.where(sels[1], recs[1], 0.0)
    wgt = wgt + jnp.where(sels[2], recs[2], 0.0)
    wgt = wgt / rec_sum

    interp = jnp.dot(p2_ref[0], wgt, preferred_element_type=jnp.float32)
    h = (jnp.dot(w1a_ref[...], interp, preferred_element_type=jnp.float32)
         + jnp.dot(w1b_ref[...], p1_ref[0], preferred_element_type=jnp.float32)
         + b1_ref[...])
    h_ref[0] = h

    @pl.when((pl.program_id(0) == 0) & (pl.program_id(1) == 0))
    def _init():
        sum_ref[...] = jnp.zeros_like(sum_ref)
        sq_ref[...] = jnp.zeros_like(sq_ref)

    sum_ref[...] += jnp.sum(h, axis=1, keepdims=True)
    sq_ref[...] += jnp.sum(h * h, axis=1, keepdims=True)


def _stage2_body(h_ref, sc_ref, sh_ref, w2_ref, b2_ref, out_ref):
    g = jnp.maximum(h_ref[0] * sc_ref[...] + sh_ref[...], 0.0)
    o = jnp.dot(w2_ref[...], g, preferred_element_type=jnp.float32) + b2_ref[...]
    out_ref[0] = jnp.maximum(o, 0.0)


def kernel(xyz1, xyz2, points1, points2, W1, b1, gamma1, beta1, W2, b2):
    x2t = jnp.transpose(xyz2, (0, 2, 1))
    w1a = W1[:, :_D2]
    w1b = W1[:, _D2:]
    b1c = b1[:, None]

    h, hsum, hsq = pl.pallas_call(
        _stage1_body,
        grid=(_B, _NB),
        in_specs=[
            pl.BlockSpec((1, _S, 3), lambda b, n: (b, 0, 0)),
            pl.BlockSpec((1, 3, _BLK), lambda b, n: (b, 0, n)),
            pl.BlockSpec((1, _D2, _S), lambda b, n: (b, 0, 0)),
            pl.BlockSpec((1, _D1, _BLK), lambda b, n: (b, 0, n)),
            pl.BlockSpec((_M0, _D2), lambda b, n: (0, 0)),
            pl.BlockSpec((_M0, _D1), lambda b, n: (0, 0)),
            pl.BlockSpec((_M0, 1), lambda b, n: (0, 0)),
        ],
        out_specs=[
            pl.BlockSpec((1, _M0, _BLK), lambda b, n: (b, 0, n)),
            pl.BlockSpec((_M0, 1), lambda b, n: (0, 0)),
            pl.BlockSpec((_M0, 1), lambda b, n: (0, 0)),
        ],
        out_shape=[
            jax.ShapeDtypeStruct((_B, _M0, _N), jnp.float32),
            jax.ShapeDtypeStruct((_M0, 1), jnp.float32),
            jax.ShapeDtypeStruct((_M0, 1), jnp.float32),
        ],
    )(x2t, xyz1, points2, points1, w1a, w1b, b1c)

    cnt = float(_B * _N)
    mean = hsum / cnt
    var = hsq / cnt - mean * mean
    scale = gamma1[:, None] * lax.rsqrt(var + 1e-5)
    shift = beta1[:, None] - mean * scale

    out = pl.pallas_call(
        _stage2_body,
        grid=(_B, _NB),
        in_specs=[
            pl.BlockSpec((1, _M0, _BLK), lambda b, n: (b, 0, n)),
            pl.BlockSpec((_M0, 1), lambda b, n: (0, 0)),
            pl.BlockSpec((_M0, 1), lambda b, n: (0, 0)),
            pl.BlockSpec((_M1, _M0), lambda b, n: (0, 0)),
            pl.BlockSpec((_M1, 1), lambda b, n: (0, 0)),
        ],
        out_specs=pl.BlockSpec((1, _M1, _BLK), lambda b, n: (b, 0, n)),
        out_shape=jax.ShapeDtypeStruct((_B, _M1, _N), jnp.float32),
    )(h, scale, shift, W2, b2[:, None])
    return out


# speedup vs baseline: 28.3226x; 1.1967x over previous
import jax
import jax.numpy as jnp
from jax import lax
from jax.experimental import pallas as pl

_B, _N, _S, _D1, _D2 = 16, 4096, 1024, 256, 512
_CIN = _D1 + _D2
_M0, _M1 = 512, 512
_BLK = 512
_NB = _N // _BLK
_BLK2 = 1024
_NB2 = _N // _BLK2


def _stage1_body(x2t_ref, x1_ref, p2_ref, p1_ref, w1a_ref, w1b_ref,
                 h_ref, sum_ref, sq_ref):
    x2t = x2t_ref[0]
    x1b = x1_ref[0]
    n2 = 0.25 * jnp.sum(x2t * x2t, axis=1, keepdims=True)
    n1 = jnp.sum(x1b * x1b, axis=0, keepdims=True)
    d = jnp.dot(x2t, x1b, preferred_element_type=jnp.float32) + (n2 + n1)

    m1 = jnp.min(d, axis=0, keepdims=True)
    m2 = jnp.min(jnp.where(d <= m1, jnp.float32(jnp.inf), d), axis=0,
                 keepdims=True)
    m3 = jnp.min(jnp.where(d <= m2, jnp.float32(jnp.inf), d), axis=0,
                 keepdims=True)
    w0 = jnp.where(d <= m3, 1.0 / (d + 1e-8), 0.0)
    rs = jnp.sum(w0, axis=0, keepdims=True)
    wgt = (w0 * (1.0 / rs)).astype(jnp.bfloat16)

    interp = jnp.dot(p2_ref[0], wgt, preferred_element_type=jnp.float32)
    h = jnp.dot(w1a_ref[...], interp.astype(jnp.bfloat16),
                preferred_element_type=jnp.float32)
    h = h + jnp.dot(w1b_ref[...], p1_ref[0],
                    preferred_element_type=jnp.float32)
    h_ref[0] = h.astype(jnp.bfloat16)

    @pl.when((pl.program_id(0) == 0) & (pl.program_id(1) == 0))
    def _init():
        sum_ref[...] = jnp.zeros_like(sum_ref)
        sq_ref[...] = jnp.zeros_like(sq_ref)

    sum_ref[...] += jnp.sum(h, axis=1, keepdims=True)
    sq_ref[...] += jnp.sum(h * h, axis=1, keepdims=True)


def _stage2_body(h_ref, sum_ref, sq_ref, g1_ref, be_ref, w2_ref, b2_ref,
                 out_ref):
    inv_cnt = 1.0 / (_B * _N)
    mean = sum_ref[...] * inv_cnt
    var = sq_ref[...] * inv_cnt - mean * mean
    scale = g1_ref[...] * lax.rsqrt(var + 1e-5)
    shift = be_ref[...] - mean * scale
    g = jnp.maximum(h_ref[0].astype(jnp.float32) * scale + shift, 0.0)
    o = jnp.dot(w2_ref[...], g.astype(jnp.bfloat16),
                preferred_element_type=jnp.float32) + b2_ref[...]
    out_ref[0] = jnp.maximum(o, 0.0)


def kernel(xyz1, xyz2, points1, points2, W1, b1, gamma1, beta1, W2, b2):
    del b1
    x2t = jnp.transpose(xyz2, (0, 2, 1)) * (-2.0)
    p2b = points2.astype(jnp.bfloat16)
    p1b = points1.astype(jnp.bfloat16)
    w1a = W1[:, :_D2].astype(jnp.bfloat16)
    w1b = W1[:, _D2:].astype(jnp.bfloat16)
    w2c = W2.astype(jnp.bfloat16)

    h, hsum, hsq = pl.pallas_call(
        _stage1_body,
        grid=(_B, _NB),
        in_specs=[
            pl.BlockSpec((1, _S, 3), lambda b, n: (b, 0, 0)),
            pl.BlockSpec((1, 3, _BLK), lambda b, n: (b, 0, n)),
            pl.BlockSpec((1, _D2, _S), lambda b, n: (b, 0, 0)),
            pl.BlockSpec((1, _D1, _BLK), lambda b, n: (b, 0, n)),
            pl.BlockSpec((_M0, _D2), lambda b, n: (0, 0)),
            pl.BlockSpec((_M0, _D1), lambda b, n: (0, 0)),
        ],
        out_specs=[
            pl.BlockSpec((1, _M0, _BLK), lambda b, n: (b, 0, n)),
            pl.BlockSpec((_M0, 1), lambda b, n: (0, 0)),
            pl.BlockSpec((_M0, 1), lambda b, n: (0, 0)),
        ],
        out_shape=[
            jax.ShapeDtypeStruct((_B, _M0, _N), jnp.bfloat16),
            jax.ShapeDtypeStruct((_M0, 1), jnp.float32),
            jax.ShapeDtypeStruct((_M0, 1), jnp.float32),
        ],
    )(x2t, xyz1, p2b, p1b, w1a, w1b)

    out = pl.pallas_call(
        _stage2_body,
        grid=(_B, _NB2),
        in_specs=[
            pl.BlockSpec((1, _M0, _BLK2), lambda b, n: (b, 0, n)),
            pl.BlockSpec((_M0, 1), lambda b, n: (0, 0)),
            pl.BlockSpec((_M0, 1), lambda b, n: (0, 0)),
            pl.BlockSpec((_M0, 1), lambda b, n: (0, 0)),
            pl.BlockSpec((_M0, 1), lambda b, n: (0, 0)),
            pl.BlockSpec((_M1, _M0), lambda b, n: (0, 0)),
            pl.BlockSpec((_M1, 1), lambda b, n: (0, 0)),
        ],
        out_specs=pl.BlockSpec((1, _M1, _BLK2), lambda b, n: (b, 0, n)),
        out_shape=jax.ShapeDtypeStruct((_B, _M1, _N), jnp.float32),
    )(h, hsum, hsq, gamma1[:, None], beta1[:, None], w2c, b2[:, None])
    return out


# speedup vs baseline: 31.2326x; 1.1027x over previous
import jax
import jax.numpy as jnp
from jax import lax
from jax.experimental import pallas as pl

_B, _N, _S, _D1, _D2 = 16, 4096, 1024, 256, 512
_CIN = _D1 + _D2
_M0, _M1 = 512, 512
_BLK = 512
_NB = _N // _BLK
_BLK2 = 1024
_NB2 = _N // _BLK2


def _stage1_body(x2t_ref, x1_ref, p2_ref, p1_ref, w1a_ref, w1b_ref,
                 h_ref, sum_ref, sq_ref):
    x2t = x2t_ref[0]
    x1b = x1_ref[0]
    n2 = 0.25 * jnp.sum(x2t * x2t, axis=1, keepdims=True)
    n1 = jnp.sum(x1b * x1b, axis=0, keepdims=True)
    d = jnp.dot(x2t, x1b, preferred_element_type=jnp.float32) + (n2 + n1)

    m1 = jnp.min(d, axis=0, keepdims=True)
    m2 = jnp.min(jnp.where(d <= m1, jnp.float32(jnp.inf), d), axis=0,
                 keepdims=True)
    m3 = jnp.min(jnp.where(d <= m2, jnp.float32(jnp.inf), d), axis=0,
                 keepdims=True)
    w0 = jnp.where(d <= m3, 1.0 / (d + 1e-8), 0.0)
    rs = jnp.sum(w0, axis=0, keepdims=True)
    wgt = (w0 * (1.0 / rs)).astype(jnp.bfloat16)

    interp = jnp.dot(p2_ref[0].astype(jnp.bfloat16), wgt,
                     preferred_element_type=jnp.float32)
    h = jnp.dot(w1a_ref[...], interp.astype(jnp.bfloat16),
                preferred_element_type=jnp.float32)
    h = h + jnp.dot(w1b_ref[...], p1_ref[0].astype(jnp.bfloat16),
                    preferred_element_type=jnp.float32)
    h_ref[0] = h.astype(jnp.bfloat16)

    @pl.when((pl.program_id(0) == 0) & (pl.program_id(1) == 0))
    def _init():
        sum_ref[...] = jnp.zeros_like(sum_ref)
        sq_ref[...] = jnp.zeros_like(sq_ref)

    sum_ref[...] += jnp.sum(h, axis=1, keepdims=True)
    sq_ref[...] += jnp.sum(h * h, axis=1, keepdims=True)


def _stage2_body(h_ref, sum_ref, sq_ref, g1_ref, be_ref, w2_ref, b2_ref,
                 out_ref):
    inv_cnt = 1.0 / (_B * _N)
    mean = sum_ref[...] * inv_cnt
    var = sq_ref[...] * inv_cnt - mean * mean
    scale = g1_ref[...] * lax.rsqrt(var + 1e-5)
    shift = be_ref[...] - mean * scale
    g = jnp.maximum(h_ref[0].astype(jnp.float32) * scale + shift, 0.0)
    o = jnp.dot(w2_ref[...], g.astype(jnp.bfloat16),
                preferred_element_type=jnp.float32) + b2_ref[...]
    out_ref[0] = jnp.maximum(o, 0.0)


def kernel(xyz1, xyz2, points1, points2, W1, b1, gamma1, beta1, W2, b2):
    del b1
    x2t = jnp.transpose(xyz2, (0, 2, 1)) * (-2.0)
    w1a = W1[:, :_D2].astype(jnp.bfloat16)
    w1b = W1[:, _D2:].astype(jnp.bfloat16)

    h, hsum, hsq = pl.pallas_call(
        _stage1_body,
        grid=(_B, _NB),
        in_specs=[
            pl.BlockSpec((1, _S, 3), lambda b, n: (b, 0, 0)),
            pl.BlockSpec((1, 3, _BLK), lambda b, n: (b, 0, n)),
            pl.BlockSpec((1, _D2, _S), lambda b, n: (b, 0, 0)),
            pl.BlockSpec((1, _D1, _BLK), lambda b, n: (b, 0, n)),
            pl.BlockSpec((_M0, _D2), lambda b, n: (0, 0)),
            pl.BlockSpec((_M0, _D1), lambda b, n: (0, 0)),
        ],
        out_specs=[
            pl.BlockSpec((1, _M0, _BLK), lambda b, n: (b, 0, n)),
            pl.BlockSpec((_M0, 1), lambda b, n: (0, 0)),
            pl.BlockSpec((_M0, 1), lambda b, n: (0, 0)),
        ],
        out_shape=[
            jax.ShapeDtypeStruct((_B, _M0, _N), jnp.bfloat16),
            jax.ShapeDtypeStruct((_M0, 1), jnp.float32),
            jax.ShapeDtypeStruct((_M0, 1), jnp.float32),
        ],
    )(x2t, xyz1, points2, points1, w1a, w1b)

    out = pl.pallas_call(
        _stage2_body,
        grid=(_B, _NB2),
        in_specs=[
            pl.BlockSpec((1, _M0, _BLK2), lambda b, n: (b, 0, n)),
            pl.BlockSpec((_M0, 1), lambda b, n: (0, 0)),
            pl.BlockSpec((_M0, 1), lambda b, n: (0, 0)),
            pl.BlockSpec((_M0, 1), lambda b, n: (0, 0)),
            pl.BlockSpec((_M0, 1), lambda b, n: (0, 0)),
            pl.BlockSpec((_M1, _M0), lambda b, n: (0, 0)),
            pl.BlockSpec((_M1, 1), lambda b, n: (0, 0)),
        ],
        out_specs=pl.BlockSpec((1, _M1, _BLK2), lambda b, n: (b, 0, n)),
        out_shape=jax.ShapeDtypeStruct((_B, _M1, _N), jnp.float32),
    )(h, hsum, hsq, gamma1[:, None], beta1[:, None],
      W2.astype(jnp.bfloat16), b2[:, None])
    return out
